# baseline (speedup 1.0000x reference)
import functools
import math

import jax
import jax.numpy as jnp
from jax import lax
from jax.experimental import pallas as pl
from jax.experimental.pallas import tpu as pltpu
from jax.experimental.pallas import tpu_sc as plsc

_BATCH = 16384
_DIM = 64
_NCLS = 1000000
_NSAMP = 100
_NPAD = 128
_NC = 2
_NSUB = 16
_NW = _NC * _NSUB
_BPAD = 16640
_BPW = _BPAD // _NW
_CHUNK = 104
_NCHUNK = _BPW // _CHUNK
_BGRAN = 16

_BLK = 2048
_GRID = _BATCH // _BLK


def _sc_gather(projection, bias2d, ids2d, ids16_2d):
    mesh = plsc.VectorSubcoreMesh(core_axis_name="c", subcore_axis_name="s")

    @functools.partial(
        pl.kernel,
        mesh=mesh,
        compiler_params=pltpu.CompilerParams(use_tc_tiling_on_sc=False),
        out_type=(
            jax.ShapeDtypeStruct((_BPAD, _DIM), jnp.float32),
            jax.ShapeDtypeStruct((_BPAD, _BGRAN), jnp.float32),
        ),
        scratch_types=[
            pltpu.VMEM((_NCHUNK, _CHUNK), jnp.int32),
            pltpu.VMEM((_NCHUNK, _CHUNK), jnp.int32),
            pltpu.VMEM((_BPW, _DIM), jnp.float32),
            pltpu.VMEM((_BPW, _BGRAN), jnp.float32),
            pltpu.SemaphoreType.DMA,
            pltpu.SemaphoreType.DMA,
        ],
    )
    def gather_kernel(proj_hbm, bias_hbm, ids_hbm, ids16_hbm,
                      rows_out, brows_out,
                      idx_v, idx16_v, rows_v, brows_v, sem_r, sem_b):
        wid = lax.axis_index("s") * _NC + lax.axis_index("c")
        pltpu.sync_copy(ids_hbm.at[wid], idx_v)
        pltpu.sync_copy(ids16_hbm.at[wid], idx16_v)
        copies = []
        for c in range(_NCHUNK):
            copies.append(pltpu.async_copy(
                proj_hbm.at[idx_v.at[c]],
                rows_v.at[pl.ds(c * _CHUNK, _CHUNK)], sem_r))
            copies.append(pltpu.async_copy(
                bias_hbm.at[idx16_v.at[c]],
                brows_v.at[pl.ds(c * _CHUNK, _CHUNK)], sem_b))
        for h in copies:
            h.wait()
        base = wid * _BPW
        pltpu.sync_copy(rows_v, rows_out.at[pl.ds(base, _BPW)])
        pltpu.sync_copy(brows_v, brows_out.at[pl.ds(base, _BPW)])

    return gather_kernel(projection, bias2d, ids2d, ids16_2d)


def _tc_loss_body(pred_ref, rows_ref, brow_ref, lab_ref, ps_ref, bs_ref,
                  samp_ref, out_ref):
    i = pl.program_id(0)

    @pl.when(i == 0)
    def _init():
        out_ref[...] = jnp.zeros((1, 1), jnp.float32)

    pred = pred_ref[...]
    rows = rows_ref[...]
    brow = brow_ref[...]
    lab = lab_ref[...]
    ps = ps_ref[...]
    bs_row = bs_ref[...]
    samp = samp_ref[...]

    iota16 = lax.broadcasted_iota(jnp.int32, (_BLK, _BGRAN), 1)
    rem = lab & (_BGRAN - 1)
    bias_l = jnp.sum(jnp.where(iota16 == rem, brow, 0.0), axis=1,
                     keepdims=True)
    ls = jnp.sum(pred * rows, axis=1, keepdims=True) + bias_l

    noise = lax.dot_general(pred, ps, (((1,), (1,)), ((), ())),
                            preferred_element_type=jnp.float32)
    noise = noise + bs_row
    rej = (lab == samp)
    nrej = jnp.sum(rej.astype(jnp.float32), axis=1, keepdims=True)
    noise = noise - 1e6 * rej.astype(jnp.float32)
    noise = noise - jnp.log(float(_NSAMP) - nrej)
    col = lax.broadcasted_iota(jnp.int32, (_BLK, _NPAD), 1)
    noise = jnp.where(col < _NSAMP, noise, -1e30)

    m = jnp.maximum(ls, jnp.max(noise, axis=1, keepdims=True))
    se = jnp.exp(ls - m) + jnp.sum(jnp.exp(noise - m), axis=1, keepdims=True)
    nll = m + jnp.log(se) - ls
    out_ref[...] += jnp.sum(nll, axis=(0, 1), keepdims=True)

    @pl.when(i == _GRID - 1)
    def _fin():
        out_ref[...] = out_ref[...] / float(_BATCH)


def kernel(predictions, labels, projection, bias, samples):
    npad_ids = _BPAD - (_BATCH + _NSAMP)
    ids = jnp.concatenate(
        [labels, samples, jnp.zeros((npad_ids,), jnp.int32)])
    ids16 = lax.shift_right_logical(ids, 4)
    ids2d = ids.reshape(_NW, _NCHUNK, _CHUNK)
    ids16_2d = ids16.reshape(_NW, _NCHUNK, _CHUNK)
    bias2d = bias.reshape(_NCLS // _BGRAN, _BGRAN)

    rows, brows = _sc_gather(projection, bias2d, ids2d, ids16_2d)

    rows_l = lax.slice(rows, (0, 0), (_BATCH, _DIM))
    brow_l = lax.slice(brows, (0, 0), (_BATCH, _BGRAN))
    ps = lax.slice(rows, (_BATCH, 0), (_BATCH + _NPAD, _DIM))
    brow_s = lax.slice(brows, (_BATCH, 0), (_BATCH + _NPAD, _BGRAN))

    samp_pad = jnp.concatenate(
        [samples, jnp.full((_NPAD - _NSAMP,), -1, jnp.int32)])
    bias_s = jnp.take_along_axis(
        brow_s, (samp_pad & (_BGRAN - 1))[:, None], axis=1)[:, 0]
    bs_row = (bias_s + math.log(_NCLS - 1)).reshape(1, _NPAD)
    samp2d = samp_pad.reshape(1, _NPAD)
    lab2d = labels.reshape(_BATCH, 1)

    out = pl.pallas_call(
        _tc_loss_body,
        grid=(_GRID,),
        in_specs=[
            pl.BlockSpec((_BLK, _DIM), lambda i: (i, 0)),
            pl.BlockSpec((_BLK, _DIM), lambda i: (i, 0)),
            pl.BlockSpec((_BLK, _BGRAN), lambda i: (i, 0)),
            pl.BlockSpec((_BLK, 1), lambda i: (i, 0)),
            pl.BlockSpec((_NPAD, _DIM), lambda i: (0, 0)),
            pl.BlockSpec((1, _NPAD), lambda i: (0, 0)),
            pl.BlockSpec((1, _NPAD), lambda i: (0, 0)),
        ],
        out_specs=pl.BlockSpec((1, 1), lambda i: (0, 0)),
        out_shape=jax.ShapeDtypeStruct((1, 1), jnp.float32),
    )(predictions, rows_l, brow_l, lab2d, ps, bs_row, samp2d)
    return out[0, 0]


# speedup vs baseline: 1.0065x; 1.0065x over previous
import dataclasses
import functools
import math

import jax
import jax.numpy as jnp
from jax import lax
from jax.experimental import pallas as pl
from jax.experimental.pallas import tpu as pltpu
from jax.experimental.pallas import tpu_sc as plsc

_BATCH = 16384
_DIM = 64
_NCLS = 1000000
_NSAMP = 100
_NPAD = 128
_NW = 32
_LPW = _BATCH // _NW
_BSZ = 256
_NBATCH = _LPW // _BSZ
_PROWS = _NCLS // 2
_BPAD = 1000448
_BGRAN = _BPAD // 128


def _sc_gather_scores(proj2, pred_t, bgran, ids3d, samp3d):
    mesh = plsc.VectorSubcoreMesh(core_axis_name="c", subcore_axis_name="s")
    cp = pltpu.CompilerParams()
    if "needs_layout_passes" in pltpu.CompilerParams.__dataclass_fields__:
        cp = dataclasses.replace(cp, needs_layout_passes=False)

    @functools.partial(
        pl.kernel,
        mesh=mesh,
        compiler_params=cp,
        out_type=(
            jax.ShapeDtypeStruct((_NW, 1, _LPW), jnp.float32),
            jax.ShapeDtypeStruct((_DIM, _NPAD), jnp.float32),
            jax.ShapeDtypeStruct((1, _NPAD), jnp.float32),
        ),
        scratch_types=[
            pltpu.VMEM((1, _LPW), jnp.int32),
            pltpu.VMEM((4, 128), jnp.int32),
            pltpu.VMEM((_BSZ, 128), jnp.float32),
            pltpu.VMEM((4, 128), jnp.int32),
            pltpu.VMEM((_BSZ, 128), jnp.float32),
            pltpu.VMEM((_DIM, _LPW), jnp.float32),
            pltpu.VMEM((1, _LPW), jnp.float32),
            pltpu.VMEM((1, _NPAD), jnp.int32),
            pltpu.VMEM((_DIM, _NPAD), jnp.float32),
            pltpu.VMEM((1, _NPAD), jnp.float32),
            pltpu.SemaphoreType.DMA,
            pltpu.SemaphoreType.DMA,
        ],
    )
    def k(proj_hbm, pred_hbm, bgran_hbm, ids_hbm, samp_hbm,
          ls_out, ps_out, bs_out,
          ids_v, idxbuf, rowbuf, bidx, biasbuf, pred_local, ls_local,
          samp_v, ps_local, bs_local, sem_g, sem_m):
        wid = lax.axis_index("s") * 2 + lax.axis_index("c")
        pltpu.sync_copy(ids_hbm.at[wid], ids_v)

        @pl.loop(0, _LPW // 16)
        def _mkidx(g):
            idv = ids_v[0, pl.ds(16 * g, 16)]
            row = g >> 3
            col = 16 * (g & 7)
            idxbuf[row, pl.ds(col, 16)] = idv >> 1
            bidx[row, pl.ds(col, 16)] = idv >> 7

        pltpu.sync_copy(
            pred_hbm.at[:, pl.ds(pl.multiple_of(wid * _LPW, _LPW), _LPW)],
            pred_local)

        @pl.loop(0, _NBATCH)
        def _batch(t):
            c0 = t * (_BSZ // 128)
            for c in range(_BSZ // 128):
                pltpu.async_copy(proj_hbm.at[idxbuf.at[c0 + c]],
                                 rowbuf.at[pl.ds(128 * c, 128)], sem_g)
                pltpu.async_copy(bgran_hbm.at[bidx.at[c0 + c]],
                                 biasbuf.at[pl.ds(128 * c, 128)], sem_m)
            for c in range(_BSZ // 128):
                pltpu.make_async_copy(proj_hbm.at[idxbuf.at[c0 + c]],
                                      rowbuf.at[pl.ds(128 * c, 128)],
                                      sem_g).wait()
                pltpu.make_async_copy(bgran_hbm.at[bidx.at[c0 + c]],
                                      biasbuf.at[pl.ds(128 * c, 128)],
                                      sem_m).wait()

            @pl.loop(0, _BSZ // 16)
            def _dot(g):
                j0 = _BSZ * t + 16 * g
                idv = ids_v[0, pl.ds(j0, 16)]
                rowv = 16 * g + lax.iota(jnp.int32, 16)
                acc = plsc.load_gather(biasbuf, [rowv, idv & 127])
                half = (idv & 1) << 6
                for d in range(_DIM):
                    val = plsc.load_gather(rowbuf, [rowv, half + d])
                    acc += val * pred_local[d, pl.ds(j0, 16)]
                ls_local[0, pl.ds(j0, 16)] = acc
        pltpu.sync_copy(ls_local, ls_out.at[wid])

        @pl.when(wid == _NW - 1)
        def _samples():
            pltpu.sync_copy(samp_hbm.at[0], samp_v)

            @pl.loop(0, _NPAD // 16)
            def _mksidx(g):
                sv = samp_v[0, pl.ds(16 * g, 16)]
                idxbuf[0, pl.ds(16 * g, 16)] = sv >> 1
                bidx[0, pl.ds(16 * g, 16)] = sv >> 7
            pltpu.async_copy(bgran_hbm.at[bidx.at[0]],
                             biasbuf.at[pl.ds(0, _NPAD)], sem_m)
            pltpu.async_copy(proj_hbm.at[idxbuf.at[0]],
                             rowbuf.at[pl.ds(0, _NPAD)], sem_g)
            pltpu.make_async_copy(proj_hbm.at[idxbuf.at[0]],
                                  rowbuf.at[pl.ds(0, _NPAD)], sem_g).wait()
            pltpu.make_async_copy(bgran_hbm.at[bidx.at[0]],
                                  biasbuf.at[pl.ds(0, _NPAD)], sem_m).wait()

            @pl.loop(0, _NPAD // 16)
            def _sx(g):
                sv = samp_v[0, pl.ds(16 * g, 16)]
                rowv = 16 * g + lax.iota(jnp.int32, 16)
                half = (sv & 1) << 6
                bs_local[0, pl.ds(16 * g, 16)] = plsc.load_gather(
                    biasbuf, [rowv, sv & 127])
                for d in range(_DIM):
                    ps_local[d, pl.ds(16 * g, 16)] = plsc.load_gather(
                        rowbuf, [rowv, half + d])
            pltpu.sync_copy(ps_local, ps_out)
            pltpu.sync_copy(bs_local, bs_out)

    return k(proj2, pred_t, bgran, ids3d, samp3d)


def _tc_loss_body(pred_ref, ls_ref, lab_ref, ps_ref, bs_ref, samp_ref,
                  out_ref):
    i = pl.program_id(0)

    @pl.when(i == 0)
    def _init():
        out_ref[...] = jnp.zeros((1, 1), jnp.float32)

    pred = pred_ref[...]
    ps = ps_ref[...]
    bs_col = bs_ref[...]
    samp = samp_ref[...]
    lab = lab_ref[0]
    ls = ls_ref[0]

    noise = lax.dot_general(ps, pred, (((0,), (0,)), ((), ())),
                            preferred_element_type=jnp.float32)
    noise = noise + bs_col
    rej = (samp == lab)
    nrej = jnp.sum(rej.astype(jnp.float32), axis=0, keepdims=True)
    noise = noise - 1e6 * rej.astype(jnp.float32)
    noise = noise - jnp.log(float(_NSAMP) - nrej)
    row = lax.broadcasted_iota(jnp.int32, (_NPAD, _LPW), 0)
    noise = jnp.where(row < _NSAMP, noise, -1e30)

    m = jnp.maximum(ls, jnp.max(noise, axis=0, keepdims=True))
    se = jnp.exp(ls - m) + jnp.sum(jnp.exp(noise - m), axis=0, keepdims=True)
    nll = m + jnp.log(se) - ls
    out_ref[...] += jnp.sum(nll, axis=(0, 1), keepdims=True)

    @pl.when(i == _NW - 1)
    def _fin():
        out_ref[...] = out_ref[...] / float(_BATCH)


def kernel(predictions, labels, projection, bias, samples):
    proj2 = projection.reshape(_PROWS, 128)
    pred_t = jnp.transpose(predictions)
    ids3d = labels.reshape(_NW, 1, _LPW)
    samp_gather = jnp.concatenate(
        [samples, jnp.zeros((_NPAD - _NSAMP,), jnp.int32)]).reshape(1, 1, _NPAD)

    bias2d = jnp.pad(bias, (0, _BPAD - _NCLS)).reshape(_BGRAN, 128)
    ls3, ps_t, bs = _sc_gather_scores(proj2, pred_t, bias2d, ids3d,
                                      samp_gather)

    bs_col = (bs.reshape(_NPAD, 1) + math.log(_NCLS - 1))
    samp_col = jnp.concatenate(
        [samples, jnp.full((_NPAD - _NSAMP,), -1, jnp.int32)]).reshape(_NPAD, 1)
    lab3 = labels.reshape(_NW, 1, _LPW)

    out = pl.pallas_call(
        _tc_loss_body,
        grid=(_NW,),
        in_specs=[
            pl.BlockSpec((_DIM, _LPW), lambda i: (0, i)),
            pl.BlockSpec((1, 1, _LPW), lambda i: (i, 0, 0)),
            pl.BlockSpec((1, 1, _LPW), lambda i: (i, 0, 0)),
            pl.BlockSpec((_DIM, _NPAD), lambda i: (0, 0)),
            pl.BlockSpec((_NPAD, 1), lambda i: (0, 0)),
            pl.BlockSpec((_NPAD, 1), lambda i: (0, 0)),
        ],
        out_specs=pl.BlockSpec((1, 1), lambda i: (0, 0)),
        out_shape=jax.ShapeDtypeStruct((1, 1), jnp.float32),
    )(pred_t, ls3, lab3, ps_t, bs_col, samp_col)
    return out[0, 0]


# speedup vs baseline: 1.0077x; 1.0011x over previous
import dataclasses
import functools
import math

import jax
import jax.numpy as jnp
from jax import lax
from jax.experimental import pallas as pl
from jax.experimental.pallas import tpu as pltpu
from jax.experimental.pallas import tpu_sc as plsc

_BATCH = 16384
_DIM = 64
_NCLS = 1000000
_NSAMP = 100
_NPAD = 128
_NW = 32
_LPW = _BATCH // _NW
_BSZ = 256
_NBATCH = _LPW // _BSZ
_PROWS = _NCLS // 2
_BPAD = 1000448
_BGRAN = _BPAD // 128


def _sc_gather_scores(proj2, pred_t, bgran, ids3d, samp3d):
    mesh = plsc.VectorSubcoreMesh(core_axis_name="c", subcore_axis_name="s")
    cp = pltpu.CompilerParams(use_tc_tiling_on_sc=False)
    if "needs_layout_passes" in pltpu.CompilerParams.__dataclass_fields__:
        cp = dataclasses.replace(cp, needs_layout_passes=False)

    @functools.partial(
        pl.kernel,
        mesh=mesh,
        compiler_params=cp,
        out_type=(
            jax.ShapeDtypeStruct((_NW, 1, _LPW), jnp.float32),
            jax.ShapeDtypeStruct((_DIM, _NPAD), jnp.float32),
            jax.ShapeDtypeStruct((1, _NPAD), jnp.float32),
        ),
        scratch_types=[
            pltpu.VMEM((1, _LPW), jnp.int32),
            pltpu.VMEM((4, 128), jnp.int32),
            pltpu.VMEM((_BSZ, _DIM), jnp.float32),
            pltpu.VMEM((4, 128), jnp.int32),
            pltpu.VMEM((_BSZ, 128), jnp.float32),
            pltpu.VMEM((_DIM, _LPW), jnp.float32),
            pltpu.VMEM((1, _LPW), jnp.float32),
            pltpu.VMEM((1, _NPAD), jnp.int32),
            pltpu.VMEM((_DIM, _NPAD), jnp.float32),
            pltpu.VMEM((1, _NPAD), jnp.float32),
            pltpu.SemaphoreType.DMA,
            pltpu.SemaphoreType.DMA,
        ],
    )
    def k(proj_hbm, pred_hbm, bgran_hbm, ids_hbm, samp_hbm,
          ls_out, ps_out, bs_out,
          ids_v, idxbuf, rowbuf, bidx, biasbuf, pred_local, ls_local,
          samp_v, ps_local, bs_local, sem_g, sem_m):
        wid = lax.axis_index("s") * 2 + lax.axis_index("c")
        pltpu.sync_copy(ids_hbm.at[wid], ids_v)

        @pl.loop(0, _LPW // 16)
        def _mkidx(g):
            idv = ids_v[0, pl.ds(16 * g, 16)]
            row = g >> 3
            col = 16 * (g & 7)
            idxbuf[row, pl.ds(col, 16)] = idv
            bidx[row, pl.ds(col, 16)] = idv >> 7

        pltpu.sync_copy(
            pred_hbm.at[:, pl.ds(pl.multiple_of(wid * _LPW, _LPW), _LPW)],
            pred_local)

        @pl.loop(0, _NBATCH)
        def _batch(t):
            c0 = t * (_BSZ // 128)
            for c in range(_BSZ // 128):
                pltpu.async_copy(proj_hbm.at[idxbuf.at[c0 + c]],
                                 rowbuf.at[pl.ds(128 * c, 128)], sem_g)
                pltpu.async_copy(bgran_hbm.at[bidx.at[c0 + c]],
                                 biasbuf.at[pl.ds(128 * c, 128)], sem_m)
            for c in range(_BSZ // 128):
                pltpu.make_async_copy(proj_hbm.at[idxbuf.at[c0 + c]],
                                      rowbuf.at[pl.ds(128 * c, 128)],
                                      sem_g).wait()
                pltpu.make_async_copy(bgran_hbm.at[bidx.at[c0 + c]],
                                      biasbuf.at[pl.ds(128 * c, 128)],
                                      sem_m).wait()

            @pl.loop(0, _BSZ // 16)
            def _dot(g):
                j0 = _BSZ * t + 16 * g
                idv = ids_v[0, pl.ds(j0, 16)]
                rowv = 16 * g + lax.iota(jnp.int32, 16)
                acc = plsc.load_gather(biasbuf, [rowv, idv & 127])
                for d in range(_DIM):
                    dv = jnp.full((16,), d, jnp.int32)
                    val = plsc.load_gather(rowbuf, [rowv, dv])
                    acc += val * pred_local[d, pl.ds(j0, 16)]
                ls_local[0, pl.ds(j0, 16)] = acc
        pltpu.sync_copy(ls_local, ls_out.at[wid])

        @pl.when(wid == _NW - 1)
        def _samples():
            pltpu.sync_copy(samp_hbm.at[0], samp_v)

            @pl.loop(0, _NPAD // 16)
            def _mksidx(g):
                sv = samp_v[0, pl.ds(16 * g, 16)]
                idxbuf[0, pl.ds(16 * g, 16)] = sv
                bidx[0, pl.ds(16 * g, 16)] = sv >> 7
            pltpu.async_copy(bgran_hbm.at[bidx.at[0]],
                             biasbuf.at[pl.ds(0, _NPAD)], sem_m)
            pltpu.async_copy(proj_hbm.at[idxbuf.at[0]],
                             rowbuf.at[pl.ds(0, _NPAD)], sem_g)
            pltpu.make_async_copy(proj_hbm.at[idxbuf.at[0]],
                                  rowbuf.at[pl.ds(0, _NPAD)], sem_g).wait()
            pltpu.make_async_copy(bgran_hbm.at[bidx.at[0]],
                                  biasbuf.at[pl.ds(0, _NPAD)], sem_m).wait()

            @pl.loop(0, _NPAD // 16)
            def _sx(g):
                sv = samp_v[0, pl.ds(16 * g, 16)]
                rowv = 16 * g + lax.iota(jnp.int32, 16)
                bs_local[0, pl.ds(16 * g, 16)] = plsc.load_gather(
                    biasbuf, [rowv, sv & 127])
                for d in range(_DIM):
                    dv = jnp.full((16,), d, jnp.int32)
                    ps_local[d, pl.ds(16 * g, 16)] = plsc.load_gather(
                        rowbuf, [rowv, dv])
            pltpu.sync_copy(ps_local, ps_out)
            pltpu.sync_copy(bs_local, bs_out)

    return k(proj2, pred_t, bgran, ids3d, samp3d)


def _tc_loss_body(pred_ref, ls_ref, lab_ref, ps_ref, bs_ref, samp_ref,
                  out_ref):
    i = pl.program_id(0)

    @pl.when(i == 0)
    def _init():
        out_ref[...] = jnp.zeros((1, 1), jnp.float32)

    pred = pred_ref[...]
    ps = ps_ref[...]
    bs_col = bs_ref[...]
    samp = samp_ref[...]
    lab = lab_ref[0]
    ls = ls_ref[0]

    noise = lax.dot_general(ps, pred, (((0,), (0,)), ((), ())),
                            preferred_element_type=jnp.float32)
    noise = noise + bs_col
    rej = (samp == lab)
    nrej = jnp.sum(rej.astype(jnp.float32), axis=0, keepdims=True)
    noise = noise - 1e6 * rej.astype(jnp.float32)
    noise = noise - jnp.log(float(_NSAMP) - nrej)
    row = lax.broadcasted_iota(jnp.int32, (_NPAD, _LPW), 0)
    noise = jnp.where(row < _NSAMP, noise, -1e30)

    m = jnp.maximum(ls, jnp.max(noise, axis=0, keepdims=True))
    se = jnp.exp(ls - m) + jnp.sum(jnp.exp(noise - m), axis=0, keepdims=True)
    nll = m + jnp.log(se) - ls
    out_ref[...] += jnp.sum(nll, axis=(0, 1), keepdims=True)

    @pl.when(i == _NW - 1)
    def _fin():
        out_ref[...] = out_ref[...] / float(_BATCH)


def kernel(predictions, labels, projection, bias, samples):
    pred_t = jnp.transpose(predictions)
    ids3d = labels.reshape(_NW, 1, _LPW)
    samp_gather = jnp.concatenate(
        [samples, jnp.zeros((_NPAD - _NSAMP,), jnp.int32)]).reshape(1, 1, _NPAD)

    bias2d = jnp.pad(bias, (0, _BPAD - _NCLS)).reshape(_BGRAN, 128)
    ls3, ps_t, bs = _sc_gather_scores(projection, pred_t, bias2d, ids3d,
                                      samp_gather)

    bs_col = (bs.reshape(_NPAD, 1) + math.log(_NCLS - 1))
    samp_col = jnp.concatenate(
        [samples, jnp.full((_NPAD - _NSAMP,), -1, jnp.int32)]).reshape(_NPAD, 1)
    lab3 = labels.reshape(_NW, 1, _LPW)

    out = pl.pallas_call(
        _tc_loss_body,
        grid=(_NW,),
        in_specs=[
            pl.BlockSpec((_DIM, _LPW), lambda i: (0, i)),
            pl.BlockSpec((1, 1, _LPW), lambda i: (i, 0, 0)),
            pl.BlockSpec((1, 1, _LPW), lambda i: (i, 0, 0)),
            pl.BlockSpec((_DIM, _NPAD), lambda i: (0, 0)),
            pl.BlockSpec((_NPAD, 1), lambda i: (0, 0)),
            pl.BlockSpec((_NPAD, 1), lambda i: (0, 0)),
        ],
        out_specs=pl.BlockSpec((1, 1), lambda i: (0, 0)),
        out_shape=jax.ShapeDtypeStruct((1, 1), jnp.float32),
    )(pred_t, ls3, lab3, ps_t, bs_col, samp_col)
    return out[0, 0]
